# baseline (speedup 1.0000x reference)
import functools

import jax
import jax.numpy as jnp
from jax import lax
from jax.experimental import pallas as pl
from jax.experimental.pallas import tpu as pltpu
from jax.experimental.pallas import tpu_sc as plsc

KM = 8
LANES = 16
SPATIAL = 1024
ST = SPATIAL // 8
SEC = 4
ST_SEC = ST // SEC
CT = 6
GROUPS = 8
ROWS_PER_CHUNK = 16
B_SC = 10

_SORT8 = [(0, 1), (2, 3), (4, 5), (6, 7), (0, 2), (1, 3), (4, 6), (5, 7),
          (1, 2), (5, 6), (0, 4), (1, 5), (2, 6), (3, 7), (2, 4), (3, 5),
          (1, 2), (3, 4), (5, 6)]
_BITONIC8 = [(0, 4), (1, 5), (2, 6), (3, 7), (0, 2), (1, 3), (4, 6), (5, 7),
             (0, 1), (2, 3), (4, 5), (6, 7)]


def _apply_network(v, net):
    v = list(v)
    for i, j in net:
        hi = jnp.maximum(v[i], v[j])
        lo = jnp.minimum(v[i], v[j])
        v[i], v[j] = hi, lo
    return v


def _merge_top8(a, b):
    c = [jnp.maximum(a[i], b[7 - i]) for i in range(KM)]
    return _apply_network(c, _BITONIC8)


def _fold_chunk16(rows, r):
    a = _apply_network(rows[:KM], _SORT8)
    b = _apply_network(rows[KM:], _SORT8)
    c = _merge_top8(a, b)
    return c if r is None else _merge_top8(list(r), c)


def _make_sc_topk():
    mesh = plsc.VectorSubcoreMesh(core_axis_name="c", subcore_axis_name="s")
    info = plsc.get_sparse_core_info()
    nc = info.num_cores
    n_units = CT * B_SC

    def div6(u):
        q = (u * 43691) >> 18
        return q, u - 6 * q

    @functools.partial(
        pl.kernel,
        out_type=jax.ShapeDtypeStruct((B_SC, CT, KM, 128), jnp.float32),
        mesh=mesh,
        scratch_types=[
            pltpu.VMEM((ST_SEC, 8, 128), jnp.float32),
            pltpu.VMEM((ST_SEC, 8, 128), jnp.float32),
            pltpu.VMEM((KM, 128), jnp.float32),
            pltpu.SemaphoreType.DMA,
            pltpu.SemaphoreType.DMA,
        ],
    )
    def topk_kernel(x_hbm, out_hbm, buf_a, buf_b, stage, sem_a, sem_b):
        w = lax.axis_index("c") * info.num_subcores + lax.axis_index("s")
        lo = (w * n_units) >> 5
        hi = ((w + 1) * n_units) >> 5
        bufs = (buf_a, buf_b)
        sems = (sem_a, sem_b)
        neg_inf = jnp.full((LANES,), -jnp.inf, jnp.float32)

        def start_dma(st0, b, ct, bi):
            pltpu.async_copy(x_hbm.at[b, pl.ds(st0, ST_SEC), ct, :, :],
                             bufs[bi], sems[bi])

        def wait_dma(bi):
            pltpu.make_async_copy(
                x_hbm.at[0, pl.ds(0, ST_SEC), 0, :, :],
                bufs[bi], sems[bi]).wait()

        b0, ct0 = div6(lo)
        start_dma(0, b0, ct0, 0)

        def per_unit(i, carry):
            u = lo + i
            b, ct = div6(u)
            for sec in range(SEC):
                bi = sec % 2
                if sec < SEC - 1:
                    start_dma((sec + 1) * ST_SEC, b, ct, 1 - bi)
                else:
                    @pl.when(i + 1 < hi - lo)
                    def _():
                        b2, ct2 = div6(u + 1)
                        start_dma(0, b2, ct2, 1 - bi)
                wait_dma(bi)
                buf = bufs[bi]

                def per_group(g, carry_g):
                    lo = g * LANES
                    if sec == 0:
                        r0 = (neg_inf,) * KM
                    else:
                        r0 = tuple(stage[k, pl.ds(lo, LANES)]
                                   for k in range(KM))

                    def per_chunk(ic, r):
                        st2 = ic * 2
                        rows = [buf[st2 + (k // 8), k % 8, pl.ds(lo, LANES)]
                                for k in range(ROWS_PER_CHUNK)]
                        return tuple(_fold_chunk16(rows, r))

                    n_chunks = (ST_SEC * 8) // ROWS_PER_CHUNK
                    r = lax.fori_loop(0, n_chunks, per_chunk, r0)
                    for k in range(KM):
                        stage[k, pl.ds(lo, LANES)] = r[k]
                    return carry_g

                lax.fori_loop(0, GROUPS, per_group, 0)
            pltpu.sync_copy(stage, out_hbm.at[b, ct])
            return carry

        lax.fori_loop(0, hi - lo, per_unit, 0)

    return topk_kernel


def _tc_column_top8(x_ref):
    r = None
    for ci in range(SPATIAL // (ROWS_PER_CHUNK * 8)):
        rows = [x_ref[0, pl.ds((ci * ROWS_PER_CHUNK + k) * 8, 8), :]
                for k in range(ROWS_PER_CHUNK)]
        r = _fold_chunk16(rows, r)
    r = list(r)
    for shift in (4, 2, 1):
        rolled = [pltpu.roll(r[7 - i], shift, 0) for i in range(KM)]
        c = [jnp.maximum(r[i], rolled[i]) for i in range(KM)]
        r = _apply_network(c, _BITONIC8)
    sub = lax.broadcasted_iota(jnp.int32, (KM, 128), 0)
    acc = r[0]
    for k in range(1, KM):
        acc = jnp.where(sub == k, r[k], acc)
    return acc


def _tc_body(*refs):
    o_ref = refs[-1]
    for bb in range(2):
        for j in range(CT):
            o_ref[bb, j] = _tc_column_top8(refs[bb * CT + j])


def _make_tc_topk(b_tc, b_off):
    def mk_spec(q):
        return pl.BlockSpec(
            (1, SPATIAL, 128),
            lambda b, q=q: (b_off + 2 * b + q // CT, 0, q % CT))
    return pl.pallas_call(
        _tc_body,
        grid=(b_tc // 2,),
        in_specs=[mk_spec(q) for q in range(2 * CT)],
        out_specs=pl.BlockSpec((2, CT, KM, 128),
                               lambda b: (b, 0, 0, 0)),
        out_shape=jax.ShapeDtypeStruct((b_tc, CT, KM, 128), jnp.float32),
    )


def kernel(x):
    B, H, W, C = x.shape
    xr = jnp.transpose(
        jnp.reshape(x, (B, H * W // 8, 8, C // 128, 128)), (0, 1, 3, 2, 4))
    out_sc = _make_sc_topk()(xr)
    x3 = jnp.reshape(x, (B, H * W, C))
    out_tc = _make_tc_topk(B - B_SC, B_SC)(*([x3] * (2 * CT)))
    out = jnp.concatenate([out_sc, out_tc], axis=0)
    out = jnp.transpose(out, (0, 1, 3, 2))
    return jnp.reshape(out, (B, KM * C))


# speedup vs baseline: 1.0024x; 1.0024x over previous
import functools

import jax
import jax.numpy as jnp
from jax import lax
from jax.experimental import pallas as pl
from jax.experimental.pallas import tpu as pltpu
from jax.experimental.pallas import tpu_sc as plsc

KM = 8
LANES = 16
SPATIAL = 1024
ST = SPATIAL // 8
SEC = 4
ST_SEC = ST // SEC
CT = 6
GROUPS = 8
ROWS_PER_CHUNK = 16
B_SC = 8

_SORT8 = [(0, 1), (2, 3), (4, 5), (6, 7), (0, 2), (1, 3), (4, 6), (5, 7),
          (1, 2), (5, 6), (0, 4), (1, 5), (2, 6), (3, 7), (2, 4), (3, 5),
          (1, 2), (3, 4), (5, 6)]
_BITONIC8 = [(0, 4), (1, 5), (2, 6), (3, 7), (0, 2), (1, 3), (4, 6), (5, 7),
             (0, 1), (2, 3), (4, 5), (6, 7)]


def _apply_network(v, net):
    v = list(v)
    for i, j in net:
        hi = jnp.maximum(v[i], v[j])
        lo = jnp.minimum(v[i], v[j])
        v[i], v[j] = hi, lo
    return v


def _merge_top8(a, b):
    c = [jnp.maximum(a[i], b[7 - i]) for i in range(KM)]
    return _apply_network(c, _BITONIC8)


def _fold_chunk16(rows, r):
    a = _apply_network(rows[:KM], _SORT8)
    b = _apply_network(rows[KM:], _SORT8)
    c = _merge_top8(a, b)
    return c if r is None else _merge_top8(list(r), c)


def _make_sc_topk():
    mesh = plsc.VectorSubcoreMesh(core_axis_name="c", subcore_axis_name="s")
    info = plsc.get_sparse_core_info()
    nc = info.num_cores
    n_units = CT * B_SC

    def div6(u):
        q = (u * 43691) >> 18
        return q, u - 6 * q

    @functools.partial(
        pl.kernel,
        out_type=jax.ShapeDtypeStruct((B_SC, CT, KM, 128), jnp.float32),
        mesh=mesh,
        scratch_types=[
            pltpu.VMEM((ST_SEC, 8, 128), jnp.float32),
            pltpu.VMEM((ST_SEC, 8, 128), jnp.float32),
            pltpu.VMEM((KM, 128), jnp.float32),
            pltpu.SemaphoreType.DMA,
            pltpu.SemaphoreType.DMA,
        ],
    )
    def topk_kernel(x_hbm, out_hbm, buf_a, buf_b, stage, sem_a, sem_b):
        w = lax.axis_index("c") * info.num_subcores + lax.axis_index("s")
        lo = (w * n_units) >> 5
        hi = ((w + 1) * n_units) >> 5
        bufs = (buf_a, buf_b)
        sems = (sem_a, sem_b)
        neg_inf = jnp.full((LANES,), -jnp.inf, jnp.float32)

        def start_dma(st0, b, ct, bi):
            pltpu.async_copy(x_hbm.at[b, pl.ds(st0, ST_SEC), ct, :, :],
                             bufs[bi], sems[bi])

        def wait_dma(bi):
            pltpu.make_async_copy(
                x_hbm.at[0, pl.ds(0, ST_SEC), 0, :, :],
                bufs[bi], sems[bi]).wait()

        b0, ct0 = div6(lo)
        start_dma(0, b0, ct0, 0)

        def per_unit(i, carry):
            u = lo + i
            b, ct = div6(u)
            for sec in range(SEC):
                bi = sec % 2
                if sec < SEC - 1:
                    start_dma((sec + 1) * ST_SEC, b, ct, 1 - bi)
                else:
                    @pl.when(i + 1 < hi - lo)
                    def _():
                        b2, ct2 = div6(u + 1)
                        start_dma(0, b2, ct2, 1 - bi)
                wait_dma(bi)
                buf = bufs[bi]

                def per_group(g, carry_g):
                    lo = g * LANES
                    if sec == 0:
                        r0 = (neg_inf,) * KM
                    else:
                        r0 = tuple(stage[k, pl.ds(lo, LANES)]
                                   for k in range(KM))

                    def per_chunk(ic, r):
                        st2 = ic * 2
                        rows = [buf[st2 + (k // 8), k % 8, pl.ds(lo, LANES)]
                                for k in range(ROWS_PER_CHUNK)]
                        return tuple(_fold_chunk16(rows, r))

                    n_chunks = (ST_SEC * 8) // ROWS_PER_CHUNK
                    r = lax.fori_loop(0, n_chunks, per_chunk, r0)
                    for k in range(KM):
                        stage[k, pl.ds(lo, LANES)] = r[k]
                    return carry_g

                lax.fori_loop(0, GROUPS, per_group, 0)
            pltpu.sync_copy(stage, out_hbm.at[b, ct])
            return carry

        lax.fori_loop(0, hi - lo, per_unit, 0)

    return topk_kernel


def _tc_column_top8(x_ref):
    r = None
    for ci in range(SPATIAL // (ROWS_PER_CHUNK * 8)):
        rows = [x_ref[0, pl.ds((ci * ROWS_PER_CHUNK + k) * 8, 8), :]
                for k in range(ROWS_PER_CHUNK)]
        r = _fold_chunk16(rows, r)
    r = list(r)
    for shift in (4, 2, 1):
        rolled = [pltpu.roll(r[7 - i], shift, 0) for i in range(KM)]
        c = [jnp.maximum(r[i], rolled[i]) for i in range(KM)]
        r = _apply_network(c, _BITONIC8)
    sub = lax.broadcasted_iota(jnp.int32, (KM, 128), 0)
    acc = r[0]
    for k in range(1, KM):
        acc = jnp.where(sub == k, r[k], acc)
    return acc


def _tc_body(*refs):
    o_ref = refs[-1]
    for bb in range(2):
        for j in range(CT):
            o_ref[bb, j] = _tc_column_top8(refs[bb * CT + j])


def _make_tc_topk(b_tc, b_off):
    def mk_spec(q):
        return pl.BlockSpec(
            (1, SPATIAL, 128),
            lambda b, q=q: (b_off + 2 * b + q // CT, 0, q % CT))
    return pl.pallas_call(
        _tc_body,
        grid=(b_tc // 2,),
        in_specs=[mk_spec(q) for q in range(2 * CT)],
        out_specs=pl.BlockSpec((2, CT, KM, 128),
                               lambda b: (b, 0, 0, 0)),
        out_shape=jax.ShapeDtypeStruct((b_tc, CT, KM, 128), jnp.float32),
    )


def kernel(x):
    B, H, W, C = x.shape
    xr = jnp.transpose(
        jnp.reshape(x, (B, H * W // 8, 8, C // 128, 128)), (0, 1, 3, 2, 4))
    out_sc = _make_sc_topk()(xr)
    x3 = jnp.reshape(x, (B, H * W, C))
    out_tc = _make_tc_topk(B - B_SC, B_SC)(*([x3] * (2 * CT)))
    out = jnp.concatenate([out_sc, out_tc], axis=0)
    out = jnp.transpose(out, (0, 1, 3, 2))
    return jnp.reshape(out, (B, KM * C))


# speedup vs baseline: 1.0169x; 1.0144x over previous
import functools

import jax
import jax.numpy as jnp
from jax import lax
from jax.experimental import pallas as pl
from jax.experimental.pallas import tpu as pltpu
from jax.experimental.pallas import tpu_sc as plsc

KM = 8
LANES = 16
SPATIAL = 1024
ST = SPATIAL // 8
SEC = 4
ST_SEC = ST // SEC
CT = 6
GROUPS = 8
ROWS_PER_CHUNK = 16
B_SC = 8

_SORT8 = [(0, 1), (2, 3), (4, 5), (6, 7), (0, 2), (1, 3), (4, 6), (5, 7),
          (1, 2), (5, 6), (0, 4), (1, 5), (2, 6), (3, 7), (2, 4), (3, 5),
          (1, 2), (3, 4), (5, 6)]
_BITONIC8 = [(0, 4), (1, 5), (2, 6), (3, 7), (0, 2), (1, 3), (4, 6), (5, 7),
             (0, 1), (2, 3), (4, 5), (6, 7)]


def _apply_network(v, net):
    v = list(v)
    for i, j in net:
        hi = jnp.maximum(v[i], v[j])
        lo = jnp.minimum(v[i], v[j])
        v[i], v[j] = hi, lo
    return v


def _merge_top8(a, b):
    c = [jnp.maximum(a[i], b[7 - i]) for i in range(KM)]
    return _apply_network(c, _BITONIC8)


def _fold_chunk16(rows, r):
    a = _apply_network(rows[:KM], _SORT8)
    b = _apply_network(rows[KM:], _SORT8)
    c = _merge_top8(a, b)
    return c if r is None else _merge_top8(list(r), c)


def _make_sc_topk():
    mesh = plsc.VectorSubcoreMesh(core_axis_name="c", subcore_axis_name="s")
    info = plsc.get_sparse_core_info()
    nc = info.num_cores
    n_units = CT * B_SC

    def div6(u):
        q = (u * 43691) >> 18
        return q, u - 6 * q

    @functools.partial(
        pl.kernel,
        out_type=jax.ShapeDtypeStruct((B_SC, CT, KM, 128), jnp.float32),
        mesh=mesh,
        scratch_types=[
            pltpu.VMEM((ST_SEC, 8, 128), jnp.float32),
            pltpu.VMEM((ST_SEC, 8, 128), jnp.float32),
            pltpu.VMEM((KM, 128), jnp.float32),
            pltpu.SemaphoreType.DMA,
            pltpu.SemaphoreType.DMA,
        ],
    )
    def topk_kernel(x_hbm, out_hbm, buf_a, buf_b, stage, sem_a, sem_b):
        w = lax.axis_index("s") * nc + lax.axis_index("c")
        lo = (w * n_units) >> 5
        hi = ((w + 1) * n_units) >> 5
        bufs = (buf_a, buf_b)
        sems = (sem_a, sem_b)
        neg_inf = jnp.full((LANES,), -jnp.inf, jnp.float32)

        def start_dma(st0, b, ct, bi):
            pltpu.async_copy(x_hbm.at[b, pl.ds(st0, ST_SEC), ct, :, :],
                             bufs[bi], sems[bi])

        def wait_dma(bi):
            pltpu.make_async_copy(
                x_hbm.at[0, pl.ds(0, ST_SEC), 0, :, :],
                bufs[bi], sems[bi]).wait()

        b0, ct0 = div6(lo)
        start_dma(0, b0, ct0, 0)

        def per_unit(i, carry):
            u = lo + i
            b, ct = div6(u)
            for sec in range(SEC):
                bi = sec % 2
                if sec < SEC - 1:
                    start_dma((sec + 1) * ST_SEC, b, ct, 1 - bi)
                else:
                    @pl.when(i + 1 < hi - lo)
                    def _():
                        b2, ct2 = div6(u + 1)
                        start_dma(0, b2, ct2, 1 - bi)
                wait_dma(bi)
                buf = bufs[bi]

                def per_group(g, carry_g):
                    lo = g * LANES
                    if sec == 0:
                        r0 = (neg_inf,) * KM
                    else:
                        r0 = tuple(stage[k, pl.ds(lo, LANES)]
                                   for k in range(KM))

                    def per_chunk(ic, r):
                        st2 = ic * 2
                        rows = [buf[st2 + (k // 8), k % 8, pl.ds(lo, LANES)]
                                for k in range(ROWS_PER_CHUNK)]
                        return tuple(_fold_chunk16(rows, r))

                    n_chunks = (ST_SEC * 8) // ROWS_PER_CHUNK
                    r = lax.fori_loop(0, n_chunks, per_chunk, r0)
                    for k in range(KM):
                        stage[k, pl.ds(lo, LANES)] = r[k]
                    return carry_g

                lax.fori_loop(0, GROUPS, per_group, 0)
            pltpu.sync_copy(stage, out_hbm.at[b, ct])
            return carry

        lax.fori_loop(0, hi - lo, per_unit, 0)

    return topk_kernel


def _tc_column_top8(x_ref):
    r = None
    for ci in range(SPATIAL // (ROWS_PER_CHUNK * 8)):
        rows = [x_ref[0, pl.ds((ci * ROWS_PER_CHUNK + k) * 8, 8), :]
                for k in range(ROWS_PER_CHUNK)]
        r = _fold_chunk16(rows, r)
    r = list(r)
    for shift in (4, 2, 1):
        rolled = [pltpu.roll(r[7 - i], shift, 0) for i in range(KM)]
        c = [jnp.maximum(r[i], rolled[i]) for i in range(KM)]
        r = _apply_network(c, _BITONIC8)
    sub = lax.broadcasted_iota(jnp.int32, (KM, 128), 0)
    acc = r[0]
    for k in range(1, KM):
        acc = jnp.where(sub == k, r[k], acc)
    return acc


def _tc_body(*refs):
    o_ref = refs[-1]
    for bb in range(2):
        for j in range(CT):
            o_ref[bb, j] = _tc_column_top8(refs[bb * CT + j])


def _make_tc_topk(b_tc, b_off):
    def mk_spec(q):
        return pl.BlockSpec(
            (1, SPATIAL, 128),
            lambda b, q=q: (b_off + 2 * b + q // CT, 0, q % CT))
    return pl.pallas_call(
        _tc_body,
        grid=(b_tc // 2,),
        in_specs=[mk_spec(q) for q in range(2 * CT)],
        out_specs=pl.BlockSpec((2, CT, KM, 128),
                               lambda b: (b, 0, 0, 0)),
        out_shape=jax.ShapeDtypeStruct((b_tc, CT, KM, 128), jnp.float32),
    )


def kernel(x):
    B, H, W, C = x.shape
    xr = jnp.transpose(
        jnp.reshape(x, (B, H * W // 8, 8, C // 128, 128)), (0, 1, 3, 2, 4))
    out_sc = _make_sc_topk()(xr)
    x3 = jnp.reshape(x, (B, H * W, C))
    out_tc = _make_tc_topk(B - B_SC, B_SC)(*([x3] * (2 * CT)))
    out = jnp.concatenate([out_sc, out_tc], axis=0)
    out = jnp.transpose(out, (0, 1, 3, 2))
    return jnp.reshape(out, (B, KM * C))
